# baseline (speedup 1.0000x reference)
import functools
import jax
import jax.numpy as jnp
from jax import lax
from jax.experimental import pallas as pl
from jax.experimental.pallas import tpu as pltpu

N_STU = 10000
N_EXER = 10000
K = 128
E = 160000
LAYERS = 2
B = 16384


def _head_body(gstat_ref, gbias_ref, gkd_ref, gdisc_ref, kn_ref,
               w1_ref, b1_ref, w2_ref, b2_ref, w3_ref, b3_ref, out_ref):
    stu = jax.nn.sigmoid(gstat_ref[...] + gbias_ref[...])
    kdx = jax.nn.sigmoid(gkd_ref[...])
    disc = jax.nn.sigmoid(gdisc_ref[...]) * 10.0
    x = disc * (stu - kdx) * kn_ref[...]
    w1 = jnp.abs(w1_ref[...])
    h = jax.nn.sigmoid(
        lax.dot_general(x, w1, (((1,), (1,)), ((), ())),
                        preferred_element_type=jnp.float32) + b1_ref[...])
    w2 = jnp.abs(w2_ref[...])
    h = jax.nn.sigmoid(
        lax.dot_general(h, w2, (((1,), (1,)), ((), ())),
                        preferred_element_type=jnp.float32) + b2_ref[...])
    w3 = jnp.abs(w3_ref[...])
    s = jnp.sum(h * w3, axis=1, keepdims=True) + b3_ref[0, 0]
    out_ref[...] = jax.nn.sigmoid(s)


def _mlp_head(g_stat, g_bias, g_kd, g_disc, kn_emb, W1, b1, W2, b2, W3, b3):
    BB = 2048
    grid = (B // BB,)
    bspec_x = pl.BlockSpec((BB, K), lambda i: (i, 0))
    bspec_1 = pl.BlockSpec((BB, 1), lambda i: (i, 0))

    def full(shape):
        return pl.BlockSpec(shape, lambda i: tuple(0 for _ in shape))

    out = pl.pallas_call(
        _head_body,
        grid=grid,
        in_specs=[bspec_x, bspec_1, bspec_x, bspec_1, bspec_x,
                  full((256, K)), full((1, 256)),
                  full((K, 256)), full((1, K)),
                  full((1, K)), full((1, 1))],
        out_specs=pl.BlockSpec((BB, 1), lambda i: (i, 0)),
        out_shape=jax.ShapeDtypeStruct((B, 1), jnp.float32),
    )(g_stat, g_bias, g_kd, g_disc, kn_emb,
      W1, b1.reshape(1, 256), W2, b2.reshape(1, K), W3, b3.reshape(1, 1))
    return out.reshape(-1)


def _spmm(rows, cols, vals, X, n_rows):
    return jax.ops.segment_sum(vals[:, None] * X[cols], rows,
                               num_segments=n_rows)


def kernel(stu_id, exer_id, kn_emb, ui1_u, ui1_i, w1, ui0_u, ui0_i, w0,
           d_i_1, d_j_1, d_i_0, d_j_0, stu_emb, exer_emb, stu_bias, e_disc,
           W1, b1, W2, b2, W3, b3):
    eu = jnp.concatenate([ui1_u, ui0_u])
    ei = jnp.concatenate([ui1_i, ui0_i])
    ew = jnp.concatenate([w1, w0])
    d_i = d_i_1 + d_i_0
    d_j = d_j_1 + d_j_0

    stat = stu_emb
    kd = exer_emb
    stat_sum = stat
    kd_sum = kd
    for _ in range(LAYERS):
        new_stat = _spmm(eu, ei, ew, kd, N_STU) + stat * d_i
        new_kd = _spmm(ei, eu, ew, stat, N_EXER) + kd * d_j
        stat, kd = new_stat, new_kd
        stat_sum = stat_sum + stat
        kd_sum = kd_sum + kd
    stat_final = stat_sum / (LAYERS + 1)
    kd_final = kd_sum / (LAYERS + 1)

    g_stat = stat_final[stu_id]
    g_bias = stu_bias[stu_id]
    g_kd = kd_final[exer_id]
    g_disc = e_disc[exer_id]

    return _mlp_head(g_stat, g_bias, g_kd, g_disc, kn_emb,
                     W1, b1, W2, b2, W3, b3)


# speedup vs baseline: 3.9608x; 3.9608x over previous
import functools
import jax
import jax.numpy as jnp
from jax import lax
from jax.experimental import pallas as pl
from jax.experimental.pallas import tpu as pltpu
from jax.experimental.pallas import tpu_sc as plsc

N_STU = 10000
N_EXER = 10000
K = 128
E = 160000
LAYERS = 2
B = 16384

NC = 2
NS = 16
LANES = 16

E2 = 2 * E
EC = 80
EPT = E2 // NS
NBLK = 5
CPB = (EPT // EC) // NBLK
EPB = CPB * EC
N_PAD = 10240
RPT = N_PAD // NS
RC = 32
NRC = RPT // RC
KV = K // LANES


def _scale_rows_by_edge_w(rows_buf, wbuf, j):
    def body(e, _):
        wb = plsc.load_gather(wbuf, [jnp.full((LANES,), j * EC + e, jnp.int32)])
        for k in range(KV):
            sl = (e, pl.ds(k * LANES, LANES))
            rows_buf[sl] = rows_buf[sl] * wb
        return 0
    lax.fori_loop(0, EC, body, 0)


def _half(s, src_t, base_t, base0_t, d_t, rows4d, cols4d, ew_t, out_t,
          idx_rows, idx_cols, wbuf, rows_buf, dbuf, bufA, bufB, bufC, acc,
          final):
    rbase = s * RPT
    pltpu.sync_copy(d_t.at[pl.ds(rbase, RPT)], dbuf)

    for ch in range(NRC):
        r0 = rbase + ch * RC
        pltpu.sync_copy(base_t.at[pl.ds(r0, RC)], bufA)

        def ib(r, _):
            db = plsc.load_gather(
                dbuf, [jnp.full((LANES,), ch * RC + r, jnp.int32)])
            for k in range(KV):
                sl = (r, pl.ds(k * LANES, LANES))
                bufA[sl] = bufA[sl] * db
            return 0
        lax.fori_loop(0, RC, ib, 0)
        pltpu.sync_copy(bufA, acc.at[pl.ds(r0, RC)])

    plsc.subcore_barrier()

    for b in range(NBLK):
        pltpu.sync_copy(rows4d.at[s, b], idx_rows)
        pltpu.sync_copy(cols4d.at[s, b], idx_cols)
        pltpu.sync_copy(ew_t.at[pl.ds(s * EPT + b * EPB, EPB)], wbuf)

        def echunk(j, _):
            pltpu.sync_copy(src_t.at[idx_cols.at[j]], rows_buf)
            _scale_rows_by_edge_w(rows_buf, wbuf, j)
            pltpu.sync_copy(rows_buf, acc.at[idx_rows.at[j]], add=True)
            return 0
        lax.fori_loop(0, CPB, echunk, 0)

    plsc.subcore_barrier()

    for ch in range(NRC):
        r0 = rbase + ch * RC
        if not final:
            pltpu.sync_copy(acc.at[pl.ds(r0, RC)], out_t.at[pl.ds(r0, RC)])
        else:
            pltpu.sync_copy(acc.at[pl.ds(r0, RC)], bufA)
            pltpu.sync_copy(base0_t.at[pl.ds(r0, RC)], bufB)
            pltpu.sync_copy(base_t.at[pl.ds(r0, RC)], bufC)

            def db_(r, _):
                for k in range(KV):
                    sl = (r, pl.ds(k * LANES, LANES))
                    bufA[sl] = (bufA[sl] + bufB[sl] + bufC[sl]) * (1.0 / 3.0)
                return 0
            lax.fori_loop(0, RC, db_, 0)
            pltpu.sync_copy(bufA, out_t.at[pl.ds(r0, RC)])


def _make_layer(final):
    mesh = plsc.VectorSubcoreMesh(core_axis_name="c", subcore_axis_name="s")
    scratch = [
        pltpu.VMEM((CPB, EC), jnp.int32),
        pltpu.VMEM((CPB, EC), jnp.int32),
        pltpu.VMEM((EPB,), jnp.float32),
        pltpu.VMEM((EC, K), jnp.float32),
        pltpu.VMEM((RPT,), jnp.float32),
        pltpu.VMEM((RC, K), jnp.float32),
        pltpu.VMEM((RC, K), jnp.float32),
        pltpu.VMEM((RC, K), jnp.float32),
        pltpu.VMEM_SHARED((N_PAD, K), jnp.float32),
    ]
    outs = (jax.ShapeDtypeStruct((N_PAD, K), jnp.float32),
            jax.ShapeDtypeStruct((N_PAD, K), jnp.float32))

    if final:
        def body(stat_ref, kd_ref, stat0_ref, kd0_ref, eu_ref, ei_ref, ew_ref,
                 di_ref, dj_ref, ostat_ref, okd_ref,
                 idx_rows, idx_cols, wbuf, rows_buf, dbuf, bufA, bufB, bufC,
                 acc):
            c = lax.axis_index("c")
            s = lax.axis_index("s")

            @pl.when(c == 0)
            def _():
                _half(s, kd_ref, stat_ref, stat0_ref, di_ref, eu_ref, ei_ref,
                      ew_ref, ostat_ref, idx_rows, idx_cols, wbuf, rows_buf,
                      dbuf, bufA, bufB, bufC, acc, True)

            @pl.when(c == 1)
            def _():
                _half(s, stat_ref, kd_ref, kd0_ref, dj_ref, ei_ref, eu_ref,
                      ew_ref, okd_ref, idx_rows, idx_cols, wbuf, rows_buf,
                      dbuf, bufA, bufB, bufC, acc, True)
    else:
        def body(stat_ref, kd_ref, eu_ref, ei_ref, ew_ref,
                 di_ref, dj_ref, ostat_ref, okd_ref,
                 idx_rows, idx_cols, wbuf, rows_buf, dbuf, bufA, bufB, bufC,
                 acc):
            c = lax.axis_index("c")
            s = lax.axis_index("s")

            @pl.when(c == 0)
            def _():
                _half(s, kd_ref, stat_ref, None, di_ref, eu_ref, ei_ref,
                      ew_ref, ostat_ref, idx_rows, idx_cols, wbuf, rows_buf,
                      dbuf, bufA, bufB, bufC, acc, False)

            @pl.when(c == 1)
            def _():
                _half(s, stat_ref, kd_ref, None, dj_ref, ei_ref, eu_ref,
                      ew_ref, okd_ref, idx_rows, idx_cols, wbuf, rows_buf,
                      dbuf, bufA, bufB, bufC, acc, False)

    return pl.kernel(body, out_type=outs, mesh=mesh, scratch_types=scratch,
                     compiler_params=pltpu.CompilerParams(
                         needs_layout_passes=False))


_layer_mid = _make_layer(False)
_layer_final = _make_layer(True)

QPT = B // (NC * NS)
QCH = 128
NQC = QPT // QCH


def _gather_body(statf_ref, kdf_ref, bias_ref, disc_ref, sid_ref, eid_ref,
                 gstat_ref, gkd_ref, gbias_ref, gdisc_ref,
                 sidx, eidx, g1, g2, btab, dtab, sbuf, dbuf):
    c = lax.axis_index("c")
    s = lax.axis_index("s")
    wid = s * NC + c
    qb = wid * QPT
    pltpu.sync_copy(sid_ref.at[wid], sidx)
    pltpu.sync_copy(eid_ref.at[wid], eidx)
    pltpu.sync_copy(bias_ref, btab)
    pltpu.sync_copy(disc_ref, dtab)
    for t in range(NQC):
        pltpu.sync_copy(statf_ref.at[sidx.at[t]], g1)
        pltpu.sync_copy(g1, gstat_ref.at[pl.ds(qb + t * QCH, QCH)])
        pltpu.sync_copy(kdf_ref.at[eidx.at[t]], g2)
        pltpu.sync_copy(g2, gkd_ref.at[pl.ds(qb + t * QCH, QCH)])

        def sg(i, _):
            iv = sidx[t, pl.ds(i * LANES, LANES)]
            bv = plsc.load_gather(btab, [iv])
            sbuf[pl.ds(i * LANES, LANES)] = bv
            ev = eidx[t, pl.ds(i * LANES, LANES)]
            dv = plsc.load_gather(dtab, [ev])
            dbuf[pl.ds(i * LANES, LANES)] = dv
            return 0
        lax.fori_loop(0, QCH // LANES, sg, 0)
        pltpu.sync_copy(sbuf, gbias_ref.at[pl.ds(qb + t * QCH, QCH)])
        pltpu.sync_copy(dbuf, gdisc_ref.at[pl.ds(qb + t * QCH, QCH)])


_gather_kernel = pl.kernel(
    _gather_body,
    out_type=(jax.ShapeDtypeStruct((B, K), jnp.float32),
              jax.ShapeDtypeStruct((B, K), jnp.float32),
              jax.ShapeDtypeStruct((B,), jnp.float32),
              jax.ShapeDtypeStruct((B,), jnp.float32)),
    mesh=plsc.VectorSubcoreMesh(core_axis_name="c", subcore_axis_name="s"),
    compiler_params=pltpu.CompilerParams(needs_layout_passes=False),
    scratch_types=[
        pltpu.VMEM((NQC, QCH), jnp.int32),
        pltpu.VMEM((NQC, QCH), jnp.int32),
        pltpu.VMEM((QCH, K), jnp.float32),
        pltpu.VMEM((QCH, K), jnp.float32),
        pltpu.VMEM((N_STU,), jnp.float32),
        pltpu.VMEM((N_EXER,), jnp.float32),
        pltpu.VMEM((QCH,), jnp.float32),
        pltpu.VMEM((QCH,), jnp.float32),
    ],
)


def _head_body(gstat_ref, gbias_ref, gkd_ref, gdisc_ref, kn_ref,
               w1_ref, b1_ref, w2_ref, b2_ref, w3_ref, b3_ref, out_ref):
    stu = jax.nn.sigmoid(gstat_ref[...] + gbias_ref[...])
    kdx = jax.nn.sigmoid(gkd_ref[...])
    disc = jax.nn.sigmoid(gdisc_ref[...]) * 10.0
    x = disc * (stu - kdx) * kn_ref[...]
    w1 = jnp.abs(w1_ref[...])
    h = jax.nn.sigmoid(
        lax.dot_general(x, w1, (((1,), (1,)), ((), ())),
                        preferred_element_type=jnp.float32) + b1_ref[...])
    w2 = jnp.abs(w2_ref[...])
    h = jax.nn.sigmoid(
        lax.dot_general(h, w2, (((1,), (1,)), ((), ())),
                        preferred_element_type=jnp.float32) + b2_ref[...])
    w3 = jnp.abs(w3_ref[...])
    s = jnp.sum(h * w3, axis=1, keepdims=True) + b3_ref[0, 0]
    out_ref[...] = jax.nn.sigmoid(s)


def _mlp_head(g_stat, g_bias, g_kd, g_disc, kn_emb, W1, b1, W2, b2, W3, b3):
    BB = 2048
    grid = (B // BB,)
    bspec_x = pl.BlockSpec((BB, K), lambda i: (i, 0))
    bspec_1 = pl.BlockSpec((BB, 1), lambda i: (i, 0))

    def full(shape):
        return pl.BlockSpec(shape, lambda i: tuple(0 for _ in shape))

    out = pl.pallas_call(
        _head_body,
        grid=grid,
        in_specs=[bspec_x, bspec_1, bspec_x, bspec_1, bspec_x,
                  full((256, K)), full((1, 256)),
                  full((K, 256)), full((1, K)),
                  full((1, K)), full((1, 1))],
        out_specs=pl.BlockSpec((BB, 1), lambda i: (i, 0)),
        out_shape=jax.ShapeDtypeStruct((B, 1), jnp.float32),
    )(g_stat, g_bias, g_kd, g_disc, kn_emb,
      W1, b1.reshape(1, 256), W2, b2.reshape(1, K), W3, b3.reshape(1, 1))
    return out.reshape(-1)


def kernel(stu_id, exer_id, kn_emb, ui1_u, ui1_i, w1, ui0_u, ui0_i, w0,
           d_i_1, d_j_1, d_i_0, d_j_0, stu_emb, exer_emb, stu_bias, e_disc,
           W1, b1, W2, b2, W3, b3):
    eu = jnp.concatenate([ui1_u, ui0_u]).astype(jnp.int32).reshape(
        NS, NBLK, CPB, EC)
    ei = jnp.concatenate([ui1_i, ui0_i]).astype(jnp.int32).reshape(
        NS, NBLK, CPB, EC)
    ew = jnp.concatenate([w1, w0])
    pad = ((0, N_PAD - N_STU), (0, 0))
    stat0 = jnp.pad(stu_emb, pad)
    kd0 = jnp.pad(exer_emb, pad)
    d_i = jnp.pad((d_i_1 + d_i_0).reshape(-1), (0, N_PAD - N_STU))
    d_j = jnp.pad((d_j_1 + d_j_0).reshape(-1), (0, N_PAD - N_EXER))

    stat1, kd1 = _layer_mid(stat0, kd0, eu, ei, ew, d_i, d_j)
    stat_f, kd_f = _layer_final(stat1, kd1, stat0, kd0, eu, ei, ew,
                                d_i, d_j)

    sid = stu_id.astype(jnp.int32).reshape(NC * NS, NQC, QCH)
    eid = exer_id.astype(jnp.int32).reshape(NC * NS, NQC, QCH)
    g_stat, g_kd, g_bias, g_disc = _gather_kernel(
        stat_f, kd_f, stu_bias.reshape(-1), e_disc.reshape(-1), sid, eid)

    return _mlp_head(g_stat, g_bias.reshape(B, 1), g_kd, g_disc.reshape(B, 1),
                     kn_emb, W1, b1, W2, b2, W3, b3)


# speedup vs baseline: 6.8296x; 1.7243x over previous
import jax
import jax.numpy as jnp
from jax import lax
from jax.experimental import pallas as pl
from jax.experimental.pallas import tpu as pltpu
from jax.experimental.pallas import tpu_sc as plsc

N_STU = 10000
N_EXER = 10000
K = 128
E = 160000
B = 16384

NC = 2
NS = 16
LANES = 16

E2 = 2 * E
EC = 80
EPT = E2 // NS
NBLK = 10
CPB = (EPT // EC) // NBLK
EPB = CPB * EC
N_PAD = 10240
RPT = N_PAD // NS
RC = 32
NRC = RPT // RC
KV = K // LANES
NBUF = 3
UNROLL = 6


def _scale_rows_by_edge_w(rows_buf, wbuf, j):
    def body(e, _):
        wb = plsc.load_gather(wbuf, [jnp.full((LANES,), j * EC + e, jnp.int32)])
        for k in range(KV):
            sl = (e, pl.ds(k * LANES, LANES))
            rows_buf[sl] = rows_buf[sl] * wb
        return 0
    lax.fori_loop(0, EC, body, 0)


def _make_layer(final):
    mesh = plsc.VectorSubcoreMesh(core_axis_name="c", subcore_axis_name="s")
    scratch = [
        pltpu.VMEM((CPB, EC), jnp.int32),
        pltpu.VMEM((CPB, EC), jnp.int32),
        pltpu.VMEM((EPB,), jnp.float32),
        pltpu.VMEM((NBUF * EC, K), jnp.float32),
        pltpu.VMEM((RC,), jnp.float32),
        [pltpu.SemaphoreType.DMA] * NBUF,
        [pltpu.SemaphoreType.DMA] * NBUF,
        pltpu.VMEM_SHARED((N_PAD, K), jnp.float32),
    ]
    out = jax.ShapeDtypeStruct((NC, N_PAD, K), jnp.float32)

    def body(t_ref, t0_ref, edg_ref, ew_ref, d_ref, out_ref,
             idx_rows, idx_cols, wbuf, ring, dbuf, gsems, ssems, acc):
        c = lax.axis_index("c")
        o = 1 - c
        s = lax.axis_index("s")
        rbase = s * RPT
        bufA = ring.at[pl.ds(0, RC)]
        bufB = ring.at[pl.ds(EC, RC)]
        bufC = ring.at[pl.ds(2 * EC, RC)]

        for ch in range(NRC):
            r0 = rbase + ch * RC
            pltpu.sync_copy(t_ref.at[c, pl.ds(r0, RC)], bufA)
            pltpu.sync_copy(d_ref.at[c, pl.ds(r0, RC)], dbuf)

            def ib(r, _):
                db = plsc.load_gather(dbuf, [jnp.full((LANES,), r, jnp.int32)])
                for k in range(KV):
                    sl = (r, pl.ds(k * LANES, LANES))
                    bufA[sl] = bufA[sl] * db
                return 0
            lax.fori_loop(0, RC, ib, 0)
            pltpu.sync_copy(bufA, acc.at[pl.ds(r0, RC)])

        plsc.subcore_barrier()

        src_t = t_ref.at[o]
        bufs = [ring.at[pl.ds(p * EC, EC)] for p in range(NBUF)]

        def gather_start(j, p):
            pltpu.async_copy(src_t.at[idx_cols.at[j]], bufs[p], gsems[p])

        def gather_wait(j, p):
            pltpu.make_async_copy(src_t.at[idx_cols.at[j]], bufs[p],
                                  gsems[p]).wait()

        def scat_start(j, p):
            pltpu.async_copy(bufs[p], acc.at[idx_rows.at[j]], ssems[p],
                             add=True)

        def scat_wait(j, p):
            pltpu.make_async_copy(bufs[p], acc.at[idx_rows.at[j]],
                                  ssems[p]).wait()

        def step(j, p, q, guard):
            gather_wait(j, p)
            _scale_rows_by_edge_w(bufs[p], wbuf, j)
            scat_start(j, p)
            if guard:
                @pl.when(j >= 1)
                def _():
                    scat_wait(j - 1, q)

                @pl.when(j + 2 < CPB)
                def _():
                    gather_start(j + 2, q)
            else:
                scat_wait(j - 1, q)
                if j + 2 < CPB:
                    gather_start(j + 2, q)

        def block(b, _):
            pltpu.sync_copy(edg_ref.at[c, s, b], idx_rows)
            pltpu.sync_copy(edg_ref.at[o, s, b], idx_cols)
            pltpu.sync_copy(ew_ref.at[pl.ds(s * EPT + b * EPB, EPB)], wbuf)

            gather_start(0, 0)
            gather_start(1, 1)

            def group(g, _):
                for pb in range(UNROLL):
                    j = g * UNROLL + pb
                    step(j, pb % NBUF, (pb + 2) % NBUF, True)
                return 0
            lax.fori_loop(0, (CPB - 2) // UNROLL, group, 0)
            for j in range(((CPB - 2) // UNROLL) * UNROLL, CPB):
                step(j, j % NBUF, (j + 2) % NBUF, False)
            scat_wait(CPB - 1, (CPB - 1) % NBUF)
            return 0
        lax.fori_loop(0, NBLK, block, 0)

        plsc.subcore_barrier()

        for ch in range(NRC):
            r0 = rbase + ch * RC
            if not final:
                pltpu.sync_copy(acc.at[pl.ds(r0, RC)],
                                out_ref.at[c, pl.ds(r0, RC)])
            else:
                pltpu.sync_copy(acc.at[pl.ds(r0, RC)], bufA)
                pltpu.sync_copy(t0_ref.at[c, pl.ds(r0, RC)], bufB)
                pltpu.sync_copy(t_ref.at[c, pl.ds(r0, RC)], bufC)

                def db_(r, _):
                    for k in range(KV):
                        sl = (r, pl.ds(k * LANES, LANES))
                        bufA[sl] = (bufA[sl] + bufB[sl] + bufC[sl]) * (1.0 / 3.0)
                    return 0
                lax.fori_loop(0, RC, db_, 0)
                pltpu.sync_copy(bufA, out_ref.at[c, pl.ds(r0, RC)])

    return pl.kernel(body, out_type=out, mesh=mesh, scratch_types=scratch,
                     compiler_params=pltpu.CompilerParams(
                         needs_layout_passes=False))


_layer_kernel = _make_layer(False)
_layer_final_kernel = _make_layer(True)

QPT = B // (NC * NS)
QCH = 128
NQC = QPT // QCH


def _gather_body(statf_ref, kdf_ref, bias_ref, disc_ref, sid_ref, eid_ref,
                 gstat_ref, gkd_ref, gbias_ref, gdisc_ref,
                 sidx, eidx, g1, g2, btab, dtab, sbuf, dbuf):
    c = lax.axis_index("c")
    s = lax.axis_index("s")
    wid = s * NC + c
    qb = wid * QPT
    pltpu.sync_copy(sid_ref.at[wid], sidx)
    pltpu.sync_copy(eid_ref.at[wid], eidx)
    pltpu.sync_copy(bias_ref, btab)
    pltpu.sync_copy(disc_ref, dtab)
    for t in range(NQC):
        pltpu.sync_copy(statf_ref.at[sidx.at[t]], g1)
        pltpu.sync_copy(g1, gstat_ref.at[pl.ds(qb + t * QCH, QCH)])
        pltpu.sync_copy(kdf_ref.at[eidx.at[t]], g2)
        pltpu.sync_copy(g2, gkd_ref.at[pl.ds(qb + t * QCH, QCH)])

        def sg(i, _):
            iv = sidx[t, pl.ds(i * LANES, LANES)]
            bv = plsc.load_gather(btab, [iv])
            sbuf[pl.ds(i * LANES, LANES)] = bv
            ev = eidx[t, pl.ds(i * LANES, LANES)]
            dv = plsc.load_gather(dtab, [ev])
            dbuf[pl.ds(i * LANES, LANES)] = dv
            return 0
        lax.fori_loop(0, QCH // LANES, sg, 0)
        pltpu.sync_copy(sbuf, gbias_ref.at[pl.ds(qb + t * QCH, QCH)])
        pltpu.sync_copy(dbuf, gdisc_ref.at[pl.ds(qb + t * QCH, QCH)])


_gather_kernel = pl.kernel(
    _gather_body,
    out_type=(jax.ShapeDtypeStruct((B, K), jnp.float32),
              jax.ShapeDtypeStruct((B, K), jnp.float32),
              jax.ShapeDtypeStruct((B,), jnp.float32),
              jax.ShapeDtypeStruct((B,), jnp.float32)),
    mesh=plsc.VectorSubcoreMesh(core_axis_name="c", subcore_axis_name="s"),
    compiler_params=pltpu.CompilerParams(needs_layout_passes=False),
    scratch_types=[
        pltpu.VMEM((NQC, QCH), jnp.int32),
        pltpu.VMEM((NQC, QCH), jnp.int32),
        pltpu.VMEM((QCH, K), jnp.float32),
        pltpu.VMEM((QCH, K), jnp.float32),
        pltpu.VMEM((N_STU,), jnp.float32),
        pltpu.VMEM((N_EXER,), jnp.float32),
        pltpu.VMEM((QCH,), jnp.float32),
        pltpu.VMEM((QCH,), jnp.float32),
    ],
)


def _head_body(gstat_ref, gbias_ref, gkd_ref, gdisc_ref, kn_ref,
               w1_ref, b1_ref, w2_ref, b2_ref, w3_ref, b3_ref, out_ref):
    stu = jax.nn.sigmoid(gstat_ref[...] + gbias_ref[...])
    kdx = jax.nn.sigmoid(gkd_ref[...])
    disc = jax.nn.sigmoid(gdisc_ref[...]) * 10.0
    x = disc * (stu - kdx) * kn_ref[...]
    w1 = jnp.abs(w1_ref[...])
    h = jax.nn.sigmoid(
        lax.dot_general(x, w1, (((1,), (1,)), ((), ())),
                        preferred_element_type=jnp.float32) + b1_ref[...])
    w2 = jnp.abs(w2_ref[...])
    h = jax.nn.sigmoid(
        lax.dot_general(h, w2, (((1,), (1,)), ((), ())),
                        preferred_element_type=jnp.float32) + b2_ref[...])
    w3 = jnp.abs(w3_ref[...])
    sm = jnp.sum(h * w3, axis=1, keepdims=True) + b3_ref[0, 0]
    out_ref[...] = jax.nn.sigmoid(sm)


def _mlp_head(g_stat, g_bias, g_kd, g_disc, kn_emb, W1, b1, W2, b2, W3, b3):
    BB = 2048
    grid = (B // BB,)
    bspec_x = pl.BlockSpec((BB, K), lambda i: (i, 0))
    bspec_1 = pl.BlockSpec((BB, 1), lambda i: (i, 0))

    def full(shape):
        return pl.BlockSpec(shape, lambda i: tuple(0 for _ in shape))

    out = pl.pallas_call(
        _head_body,
        grid=grid,
        in_specs=[bspec_x, bspec_1, bspec_x, bspec_1, bspec_x,
                  full((256, K)), full((1, 256)),
                  full((K, 256)), full((1, K)),
                  full((1, K)), full((1, 1))],
        out_specs=pl.BlockSpec((BB, 1), lambda i: (i, 0)),
        out_shape=jax.ShapeDtypeStruct((B, 1), jnp.float32),
    )(g_stat, g_bias, g_kd, g_disc, kn_emb,
      W1, b1.reshape(1, 256), W2, b2.reshape(1, K), W3, b3.reshape(1, 1))
    return out.reshape(-1)


def kernel(stu_id, exer_id, kn_emb, ui1_u, ui1_i, w1, ui0_u, ui0_i, w0,
           d_i_1, d_j_1, d_i_0, d_j_0, stu_emb, exer_emb, stu_bias, e_disc,
           W1, b1, W2, b2, W3, b3):
    eu = jnp.concatenate([ui1_u, ui0_u]).astype(jnp.int32)
    ei = jnp.concatenate([ui1_i, ui0_i]).astype(jnp.int32)
    edg = jnp.stack([eu, ei]).reshape(NC, NS, NBLK, CPB, EC)
    ew = jnp.concatenate([w1, w0])
    pad = ((0, N_PAD - N_STU), (0, 0))
    t0 = jnp.stack([jnp.pad(stu_emb, pad), jnp.pad(exer_emb, pad)])
    d = jnp.stack([
        jnp.pad((d_i_1 + d_i_0).reshape(-1), (0, N_PAD - N_STU)),
        jnp.pad((d_j_1 + d_j_0).reshape(-1), (0, N_PAD - N_EXER))])

    t1 = _layer_kernel(t0, t0, edg, ew, d)
    tf = _layer_final_kernel(t1, t0, edg, ew, d)

    sid = stu_id.astype(jnp.int32).reshape(NC * NS, NQC, QCH)
    eid = exer_id.astype(jnp.int32).reshape(NC * NS, NQC, QCH)
    g_stat, g_kd, g_bias, g_disc = _gather_kernel(
        tf[0], tf[1], stu_bias.reshape(-1), e_disc.reshape(-1), sid, eid)

    return _mlp_head(g_stat, g_bias.reshape(B, 1), g_kd, g_disc.reshape(B, 1),
                     kn_emb, W1, b1, W2, b2, W3, b3)
